# Initial kernel scaffold; baseline (speedup 1.0000x reference)
import functools

import jax
import jax.numpy as jnp
from jax import lax
from jax.experimental import pallas as pl
from jax.experimental.pallas import tpu as pltpu
from jax.experimental.pallas import tpu_sc as plsc

_NUM_CORES = 2
_NUM_SUBCORES = 16
_NUM_WORKERS = _NUM_CORES * _NUM_SUBCORES

_CHUNK = 128


def kernel(edge, table):
    batch, seq, _ = edge.shape
    depth = table.shape[1]
    n = batch * seq * seq
    idx_flat = edge.reshape(n).astype(jnp.int32)

    per_worker = n // _NUM_WORKERS
    steps = per_worker // _CHUNK

    mesh = plsc.VectorSubcoreMesh(core_axis_name="c", subcore_axis_name="s")

    @functools.partial(
        pl.kernel,
        mesh=mesh,
        out_type=jax.ShapeDtypeStruct((n, depth), jnp.float32),
        scratch_types=[
            pltpu.VMEM((_CHUNK,), jnp.int32),
            pltpu.VMEM((_CHUNK, depth), jnp.float32),
            pltpu.SemaphoreType.DMA,
        ],
    )
    def lookup(table_hbm, idx_hbm, out_hbm, idx_v, rows_v, sem):
        wid = lax.axis_index("s") * _NUM_CORES + lax.axis_index("c")
        base = wid * per_worker

        @pl.loop(0, steps)
        def _(i):
            off = base + i * _CHUNK
            pltpu.sync_copy(idx_hbm.at[pl.ds(off, _CHUNK)], idx_v)
            pltpu.async_copy(table_hbm.at[idx_v], rows_v, sem).wait()
            pltpu.sync_copy(rows_v, out_hbm.at[pl.ds(off, _CHUNK)])

    out = lookup(table, idx_flat)
    return out.reshape(batch, seq, seq, depth)


# speedup vs baseline: 1.9937x; 1.9937x over previous
import functools

import jax
import jax.numpy as jnp
from jax import lax
from jax.experimental import pallas as pl
from jax.experimental.pallas import tpu as pltpu
from jax.experimental.pallas import tpu_sc as plsc

_NUM_CORES = 2
_NUM_SUBCORES = 16
_NUM_WORKERS = _NUM_CORES * _NUM_SUBCORES

_CHUNK = 128


def kernel(edge, table):
    batch, seq, _ = edge.shape
    depth = table.shape[1]
    n = batch * seq * seq
    idx_flat = edge.reshape(n).astype(jnp.int32)

    per_worker = n // _NUM_WORKERS
    steps = per_worker // _CHUNK

    mesh = plsc.VectorSubcoreMesh(core_axis_name="c", subcore_axis_name="s")

    @functools.partial(
        pl.kernel,
        mesh=mesh,
        out_type=jax.ShapeDtypeStruct((n, depth), jnp.float32),
        scratch_types=[
            pltpu.VMEM((_CHUNK,), jnp.int32),
            pltpu.VMEM((_CHUNK, depth), jnp.float32),
            pltpu.SemaphoreType.DMA,
        ],
        compiler_params=pltpu.CompilerParams(use_tc_tiling_on_sc=False),
    )
    def lookup(table_hbm, idx_hbm, out_hbm, idx_v, rows_v, sem):
        wid = lax.axis_index("s") * _NUM_CORES + lax.axis_index("c")
        base = wid * per_worker

        @pl.loop(0, steps)
        def _(i):
            off = base + i * _CHUNK
            pltpu.sync_copy(idx_hbm.at[pl.ds(off, _CHUNK)], idx_v)
            pltpu.async_copy(table_hbm.at[idx_v], rows_v, sem).wait()
            pltpu.sync_copy(rows_v, out_hbm.at[pl.ds(off, _CHUNK)])

    out = lookup(table, idx_flat)
    return out.reshape(batch, seq, seq, depth)


# speedup vs baseline: 2.0510x; 1.0288x over previous
import functools

import jax
import jax.numpy as jnp
from jax import lax
from jax.experimental import pallas as pl
from jax.experimental.pallas import tpu as pltpu
from jax.experimental.pallas import tpu_sc as plsc

_NUM_CORES = 2
_NUM_SUBCORES = 16
_NUM_WORKERS = _NUM_CORES * _NUM_SUBCORES

_CHUNK = 128
_ROWS = 256
_GPB = _ROWS // _CHUNK
_NBUF = 4


def kernel(edge, table):
    batch, seq, _ = edge.shape
    depth = table.shape[1]
    n = batch * seq * seq
    idx_flat = edge.reshape(n).astype(jnp.int32)

    per_worker = n // _NUM_WORKERS
    steps = per_worker // _ROWS

    mesh = plsc.VectorSubcoreMesh(core_axis_name="c", subcore_axis_name="s")

    @functools.partial(
        pl.kernel,
        mesh=mesh,
        out_type=jax.ShapeDtypeStruct((n, depth), jnp.float32),
        scratch_types=[
            pltpu.VMEM((per_worker,), jnp.int32),
            pltpu.VMEM((_NBUF * _ROWS, depth), jnp.float32),
        ]
        + [pltpu.SemaphoreType.DMA] * (2 * _NBUF),
        compiler_params=pltpu.CompilerParams(use_tc_tiling_on_sc=False),
    )
    def lookup(table_hbm, idx_hbm, out_hbm, idx_v, buf_v, *sems):
        sem_g = sems[:_NBUF]
        sem_w = sems[_NBUF:]
        wid = lax.axis_index("s") * _NUM_CORES + lax.axis_index("c")
        base = wid * per_worker

        pltpu.sync_copy(idx_hbm.at[pl.ds(base, per_worker)], idx_v)

        def issue_gathers(s, b):
            for g in range(_GPB):
                pltpu.async_copy(
                    table_hbm.at[idx_v.at[pl.ds(s * _ROWS + g * _CHUNK, _CHUNK)]],
                    buf_v.at[pl.ds(b * _ROWS + g * _CHUNK, _CHUNK)],
                    sem_g[b],
                )

        for b in range(_NBUF):
            issue_gathers(b, b)

        @pl.loop(0, steps, step=_NBUF)
        def _(i):
            for b in range(_NBUF):
                s = i + b
                pltpu.make_async_copy(
                    out_hbm.at[pl.ds(0, _ROWS)],
                    buf_v.at[pl.ds(b * _ROWS, _ROWS)],
                    sem_g[b],
                ).wait()
                write = pltpu.async_copy(
                    buf_v.at[pl.ds(b * _ROWS, _ROWS)],
                    out_hbm.at[pl.ds(base + s * _ROWS, _ROWS)],
                    sem_w[b],
                )

                @pl.when(s + _NBUF < steps)
                def _():
                    write.wait()
                    issue_gathers(s + _NBUF, b)

        for b in range(_NBUF):
            pltpu.make_async_copy(
                buf_v.at[pl.ds(b * _ROWS, _ROWS)],
                out_hbm.at[pl.ds(0, _ROWS)],
                sem_w[b],
            ).wait()

    out = lookup(table, idx_flat)
    return out.reshape(batch, seq, seq, depth)


# speedup vs baseline: 2.0530x; 1.0010x over previous
import functools

import jax
import jax.numpy as jnp
from jax import lax
from jax.experimental import pallas as pl
from jax.experimental.pallas import tpu as pltpu
from jax.experimental.pallas import tpu_sc as plsc

_NUM_CORES = 2
_NUM_SUBCORES = 16
_NUM_WORKERS = _NUM_CORES * _NUM_SUBCORES

_CHUNK = 128
_RPB = 2
_NBUF = 4


def kernel(edge, table):
    batch, seq, _ = edge.shape
    depth = table.shape[1]
    n = batch * seq * seq
    idx_flat = edge.reshape(n).astype(jnp.int32)

    per_worker = n // _NUM_WORKERS
    rows_half = seq // 2
    steps = rows_half // _RPB
    gpb = _RPB * seq // _CHUNK

    mesh = plsc.VectorSubcoreMesh(core_axis_name="c", subcore_axis_name="s")

    @functools.partial(
        pl.kernel,
        mesh=mesh,
        out_type=jax.ShapeDtypeStruct((batch, seq, seq, depth), jnp.float32),
        scratch_types=[
            pltpu.VMEM((per_worker,), jnp.int32),
            pltpu.VMEM((_NBUF * _RPB, seq, depth), jnp.float32),
        ]
        + [pltpu.SemaphoreType.DMA] * (2 * _NBUF),
        compiler_params=pltpu.CompilerParams(use_tc_tiling_on_sc=False),
    )
    def lookup(table_hbm, idx_hbm, out_hbm, idx_v, buf_v, *sems):
        sem_g = sems[:_NBUF]
        sem_w = sems[_NBUF:]
        wid = lax.axis_index("s") * _NUM_CORES + lax.axis_index("c")
        bat = wid // 2
        half = wid % 2
        base = wid * per_worker

        pltpu.sync_copy(idx_hbm.at[pl.ds(base, per_worker)], idx_v)

        def issue_gathers(s, b):
            for g in range(gpb):
                pltpu.async_copy(
                    table_hbm.at[
                        idx_v.at[pl.ds(s * _RPB * seq + g * _CHUNK, _CHUNK)]
                    ],
                    buf_v.at[b * _RPB + g],
                    sem_g[b],
                )

        for b in range(_NBUF):
            issue_gathers(b, b)

        @pl.loop(0, steps, step=_NBUF)
        def _(i):
            for b in range(_NBUF):
                s = i + b
                pltpu.make_async_copy(
                    out_hbm.at[0, pl.ds(0, _RPB)],
                    buf_v.at[pl.ds(b * _RPB, _RPB)],
                    sem_g[b],
                ).wait()
                write = pltpu.async_copy(
                    buf_v.at[pl.ds(b * _RPB, _RPB)],
                    out_hbm.at[bat, pl.ds(half * rows_half + s * _RPB, _RPB)],
                    sem_w[b],
                )

                @pl.when(s + _NBUF < steps)
                def _():
                    write.wait()
                    issue_gathers(s + _NBUF, b)

        for b in range(_NBUF):
            pltpu.make_async_copy(
                buf_v.at[pl.ds(b * _RPB, _RPB)],
                out_hbm.at[0, pl.ds(0, _RPB)],
                sem_w[b],
            ).wait()

    return lookup(table, idx_flat)


# speedup vs baseline: 4.8954x; 2.3845x over previous
import functools

import jax
import jax.numpy as jnp
from jax import lax
from jax.experimental import pallas as pl
from jax.experimental.pallas import tpu as pltpu
from jax.experimental.pallas import tpu_sc as plsc

_NUM_CORES = 2
_NUM_SUBCORES = 16
_NUM_WORKERS = _NUM_CORES * _NUM_SUBCORES

_CHUNK = 128
_RPB = 2
_NBUF = 4


def kernel(edge, table):
    batch, seq, _ = edge.shape
    depth = table.shape[1]
    n = batch * seq * seq
    idx_flat = edge.reshape(n).astype(jnp.int32)

    per_worker = n // _NUM_WORKERS
    rows_half = seq // 2
    steps = rows_half // _RPB
    gpb = _RPB * seq // _CHUNK

    mesh = plsc.VectorSubcoreMesh(core_axis_name="c", subcore_axis_name="s")

    @functools.partial(
        pl.kernel,
        mesh=mesh,
        out_type=jax.ShapeDtypeStruct((batch, seq, seq, depth), jnp.float32),
        scratch_types=[
            pltpu.VMEM((per_worker,), jnp.int32),
            pltpu.VMEM((_NBUF * _RPB, seq, depth), jnp.float32),
            pltpu.VMEM_SHARED((50, 64), jnp.float32),
        ]
        + [pltpu.SemaphoreType.DMA] * (2 * _NBUF),
        compiler_params=pltpu.CompilerParams(use_tc_tiling_on_sc=False),
    )
    def lookup(table_hbm, idx_hbm, out_hbm, idx_v, buf_v, tab_v, *sems):
        sem_g = sems[:_NBUF]
        sem_w = sems[_NBUF:]
        wid = lax.axis_index("s") * _NUM_CORES + lax.axis_index("c")
        bat = wid // 2
        half = wid % 2
        base = wid * per_worker

        @pl.when(lax.axis_index("s") == 0)
        def _():
            pltpu.sync_copy(table_hbm, tab_v)

        pltpu.sync_copy(idx_hbm.at[pl.ds(base, per_worker)], idx_v)
        plsc.subcore_barrier()

        def issue_gathers(s, b):
            for g in range(gpb):
                pltpu.async_copy(
                    tab_v.at[
                        idx_v.at[pl.ds(s * _RPB * seq + g * _CHUNK, _CHUNK)]
                    ],
                    buf_v.at[b * _RPB + g],
                    sem_g[b],
                )

        for b in range(_NBUF):
            issue_gathers(b, b)

        @pl.loop(0, steps, step=_NBUF)
        def _(i):
            for b in range(_NBUF):
                s = i + b
                pltpu.make_async_copy(
                    out_hbm.at[0, pl.ds(0, _RPB)],
                    buf_v.at[pl.ds(b * _RPB, _RPB)],
                    sem_g[b],
                ).wait()
                write = pltpu.async_copy(
                    buf_v.at[pl.ds(b * _RPB, _RPB)],
                    out_hbm.at[bat, pl.ds(half * rows_half + s * _RPB, _RPB)],
                    sem_w[b],
                )

                @pl.when(s + _NBUF < steps)
                def _():
                    write.wait()
                    issue_gathers(s + _NBUF, b)

        for b in range(_NBUF):
            pltpu.make_async_copy(
                buf_v.at[pl.ds(b * _RPB, _RPB)],
                out_hbm.at[0, pl.ds(0, _RPB)],
                sem_w[b],
            ).wait()

    return lookup(table, idx_flat)
